# Initial kernel scaffold; baseline (speedup 1.0000x reference)
import jax
import jax.numpy as jnp
from jax.experimental import pallas as pl


def kernel(mem, idx, val, sample_idx):
    raise NotImplementedError("write your pallas kernel here")


# speedup vs baseline: 4.5656x; 4.5656x over previous
import functools

import jax
import jax.numpy as jnp
from jax import lax
from jax.experimental import pallas as pl
from jax.experimental.pallas import tpu as pltpu
from jax.experimental.pallas import tpu_sc as plsc

SAMPLE_B = 1024
WRITE_B = 4096
D = 3 * 32 * 32

NC, NS = 2, 16
NW = NC * NS
ROWS_PER = SAMPLE_B // NW
CHUNK = 16
NCHUNK = ROWS_PER // CHUNK
PAD = 8


def _winner_body(idx_ref, s_ref, w_ref):
    ix = idx_ref[...]
    s = s_ref[...].reshape(1, 128)
    eq = ix == s
    j = lax.broadcasted_iota(jnp.int32, (WRITE_B, 128), 0)
    cand = jnp.where(eq, j, -1)
    w_ref[...] = jnp.max(cand, axis=0, keepdims=True).reshape(1, 1, 128)


def _winner_tc(idx, sample_idx):
    idx2 = idx.reshape(WRITE_B, 1)
    s3 = sample_idx.reshape(SAMPLE_B // 128, 1, 128)
    grid = SAMPLE_B // 128
    w = pl.pallas_call(
        _winner_body,
        grid=(grid,),
        in_specs=[
            pl.BlockSpec((WRITE_B, 1), lambda i: (0, 0)),
            pl.BlockSpec((1, 1, 128), lambda i: (i, 0, 0)),
        ],
        out_specs=pl.BlockSpec((1, 1, 128), lambda i: (i, 0, 0)),
        out_shape=jax.ShapeDtypeStruct((SAMPLE_B // 128, 1, 128), jnp.int32),
    )(idx2, s3)
    return w.reshape(SAMPLE_B)


def _sc_gather(mem2, val2, sample_idx, winner):
    mesh = plsc.VectorSubcoreMesh(core_axis_name="c", subcore_axis_name="s")

    @functools.partial(
        pl.kernel,
        mesh=mesh,
        out_type=jax.ShapeDtypeStruct((SAMPLE_B + PAD, D), jnp.float32),
        scratch_types=[
            pltpu.VMEM((ROWS_PER,), jnp.int32),
            pltpu.VMEM((ROWS_PER,), jnp.int32),
            pltpu.VMEM((CHUNK, D), jnp.float32),
            pltpu.VMEM((CHUNK, D), jnp.float32),
            pltpu.SemaphoreType.DMA,
            pltpu.SemaphoreType.DMA,
        ],
    )
    def k(mem_hbm, val_hbm, sidx_hbm, win_hbm, out_hbm,
          sidx_v, win_v, bufA, bufB, semA, semB):
        wid = lax.axis_index("s") * NC + lax.axis_index("c")
        base = wid * ROWS_PER

        pltpu.sync_copy(sidx_hbm.at[pl.ds(base, ROWS_PER)], sidx_v)
        pltpu.sync_copy(win_hbm.at[pl.ds(base, ROWS_PER)], win_v)

        dump = SAMPLE_B + (wid % PAD)
        lane = lax.broadcasted_iota(jnp.int32, (CHUNK,), 0)
        for c in range(NCHUNK):
            sidx = sidx_v[pl.ds(c * CHUNK, CHUNK)]
            w = win_v[pl.ds(c * CHUNK, CHUNK)]
            vsrc = jnp.maximum(w, 0)
            vdst = jnp.where(w >= 0, base + c * CHUNK + lane, dump)

            pltpu.async_copy(mem_hbm.at[sidx], bufA, semA).wait()
            pltpu.sync_copy(bufA, out_hbm.at[pl.ds(base + c * CHUNK, CHUNK)])
            pltpu.async_copy(val_hbm.at[vsrc], bufB, semB).wait()
            pltpu.async_copy(bufB, out_hbm.at[vdst], semB).wait()

    return k(mem2, val2, sample_idx, winner)


def kernel(mem, idx, val, sample_idx):
    cap = mem.shape[0]
    mem2 = mem.reshape(cap, D)
    val2 = val.reshape(WRITE_B, D)
    winner = _winner_tc(idx, sample_idx)
    out_ext = _sc_gather(mem2, val2, sample_idx, winner)
    return out_ext[:SAMPLE_B].reshape(SAMPLE_B, *mem.shape[1:])
